# baseline (speedup 1.0000x reference)
import functools
import numpy as np
import jax
import jax.numpy as jnp
from jax import lax
from jax.experimental import pallas as pl
from jax.experimental.pallas import tpu as pltpu
from jax.experimental.pallas import tpu_sc as plsc

_LEVELS = (8, 8, 8, 6, 5)
_CD = 5
_LANES = 128
_BM = 2048
_SUB = 256

_vals = [np.linspace(-0.5, 0.5, lv).astype(np.float32) if lv % 2 else
         (np.arange(lv) / lv - 0.5).astype(np.float32) for lv in _LEVELS]
_LO = tuple(float(v[0]) for v in _vals)
_STEP = tuple(float(v[1] - v[0]) for v in _vals)
_INV = tuple(float(1.0 / (v[1] - v[0])) for v in _vals)
_MAXI = tuple(float(lv - 1) for lv in _LEVELS)
_BASIS = tuple(int(x) for x in
               np.concatenate([[1], np.cumprod(_LEVELS[:-1])]))
_HALF = tuple(int(lv) // 2 for lv in _LEVELS)
_CA = tuple(float(2 * h * bs) for h, bs in zip(_HALF, _BASIS))
_CB = tuple(float(h * bs) for h, bs in zip(_HALF, _BASIS))


def _k1(z_ref, win_ref, bin_ref, zpt_ref, zptok_ref):
    for s in range(_BM // _SUB):
        rows = pl.ds(s * _SUB, _SUB)
        zp = jnp.dot(z_ref[rows, :], win_ref[...],
                     preferred_element_type=jnp.float32) + bin_ref[...]
        zptok_ref[rows, :] = zp[:, :8]
        for i in range(_CD):
            zpt_ref[i:i + 1, rows] = zp[:, i:i + 1].reshape(1, _SUB)


def _k2(zptok_ref, lo_ref, step_ref, inv_ref, maxi_ref, wout_ref,
        bout_ref, out_ref):
    for s in range(_BM // _SUB):
        rows = pl.ds(s * _SUB, _SUB)
        zp = zptok_ref[rows, :]
        k = jnp.clip(jnp.round((zp - lo_ref[...]) * inv_ref[...]),
                     0.0, maxi_ref[...])
        q = lo_ref[...] + k * step_ref[...]
        qb = [jnp.broadcast_to(q[:, i:i + 1], (_SUB, _LANES))
              for i in range(_CD)]
        for c in range(out_ref.shape[1] // _LANES):
            cols = pl.ds(c * _LANES, _LANES)
            acc = jnp.broadcast_to(bout_ref[0:1, cols], (_SUB, _LANES))
            for i in range(_CD):
                acc = acc + qb[i] * wout_ref[i:i + 1, cols]
            out_ref[rows, cols] = acc


def _sc_make(m):
    info = plsc.get_sparse_core_info()
    nc, ns = info.num_cores, info.num_subcores
    nw = nc * ns
    cpw = m // nw
    mesh = plsc.VectorSubcoreMesh(core_axis_name="c", subcore_axis_name="s")

    @functools.partial(
        pl.kernel, mesh=mesh,
        out_type=[
            jax.ShapeDtypeStruct((m,), jnp.float32),
            jax.ShapeDtypeStruct((nw, 16), jnp.float32),
        ],
        scratch_types=[
            pltpu.VMEM((8, cpw), jnp.float32),
            pltpu.VMEM((cpw,), jnp.float32),
            pltpu.VMEM((16,), jnp.float32),
        ],
    )
    def enc(zpt_hbm, idx_hbm, lp_hbm, zp_v, idx_v, ls_v):
        wid = lax.axis_index("s") * nc + lax.axis_index("c")
        base = wid * cpw
        pltpu.sync_copy(zpt_hbm.at[:, pl.ds(base, cpw)], zp_v)
        ls = jnp.zeros((16,), jnp.float32)
        for j in range(cpw // 16):
            sl = pl.ds(j * 16, 16)
            ia = jnp.zeros((16,), jnp.float32)
            for i in range(_CD):
                v = zp_v.at[i][sl]
                t = (v - _LO[i]) * _INV[i]
                t = jnp.clip(t, 0.0, _MAXI[i])
                k = (t + 0.5).astype(jnp.int32).astype(jnp.float32)
                q = _LO[i] + k * _STEP[i]
                ia = ia + (q * _CA[i] + _CB[i])
                e = v - q
                ls = ls + e * e
            idx_v[sl] = ia
        ls_v[...] = ls
        pltpu.sync_copy(idx_v, idx_hbm.at[pl.ds(base, cpw)])
        pltpu.sync_copy(ls_v, lp_hbm.at[wid])

    return enc


def kernel(z, W_in, b_in, W_out, b_out, v0, v1, v2, v3, v4):
    b, n, dim = z.shape
    m = b * n
    cd = _CD
    nblk = m // _BM

    win_p = jnp.zeros((dim, _LANES), jnp.float32).at[:, :cd].set(W_in.T)
    wout_p = jnp.zeros((8, dim), jnp.float32).at[:cd, :].set(W_out.T)
    bin_p = jnp.zeros((1, _LANES), jnp.float32).at[0, :cd].set(b_in)
    bout_p = b_out.reshape(1, dim)

    lo_np = np.zeros((1, 8), np.float32)
    st_np = np.zeros((1, 8), np.float32)
    iv_np = np.zeros((1, 8), np.float32)
    mx_np = np.zeros((1, 8), np.float32)
    for i in range(cd):
        lo_np[0, i] = _LO[i]
        st_np[0, i] = _STEP[i]
        iv_np[0, i] = _INV[i]
        mx_np[0, i] = _MAXI[i]

    zf = z.reshape(m, dim)
    full = lambda i: (0, 0)
    zpt, zptok = pl.pallas_call(
        _k1,
        grid=(nblk,),
        in_specs=[
            pl.BlockSpec((_BM, dim), lambda i: (i, 0)),
            pl.BlockSpec((dim, _LANES), full),
            pl.BlockSpec((1, _LANES), full),
        ],
        out_specs=[
            pl.BlockSpec((8, _BM), lambda i: (0, i)),
            pl.BlockSpec((_BM, 8), lambda i: (i, 0)),
        ],
        out_shape=[
            jax.ShapeDtypeStruct((8, m), jnp.float32),
            jax.ShapeDtypeStruct((m, 8), jnp.float32),
        ],
        compiler_params=pltpu.CompilerParams(
            dimension_semantics=("parallel",)),
    )(zf, win_p, bin_p)

    out = pl.pallas_call(
        _k2,
        grid=(nblk,),
        in_specs=[
            pl.BlockSpec((_BM, 8), lambda i: (i, 0)),
            pl.BlockSpec((1, 8), full),
            pl.BlockSpec((1, 8), full),
            pl.BlockSpec((1, 8), full),
            pl.BlockSpec((1, 8), full),
            pl.BlockSpec((8, dim), full),
            pl.BlockSpec((1, dim), full),
        ],
        out_specs=pl.BlockSpec((_BM, dim), lambda i: (i, 0)),
        out_shape=jax.ShapeDtypeStruct((m, dim), jnp.float32),
        compiler_params=pltpu.CompilerParams(
            dimension_semantics=("parallel",)),
    )(zptok, jnp.asarray(lo_np), jnp.asarray(st_np), jnp.asarray(iv_np),
      jnp.asarray(mx_np), wout_p, bout_p)

    idx, lpart = _sc_make(m)(zpt)

    out = out.reshape(b, n, dim)
    indices = idx.reshape(b, n)
    loss_val = jnp.sum(lpart) * (0.2 / (m * cd))
    return out, indices, loss_val


# speedup vs baseline: 1.0063x; 1.0063x over previous
import functools
import numpy as np
import jax
import jax.numpy as jnp
from jax import lax
from jax.experimental import pallas as pl
from jax.experimental.pallas import tpu as pltpu
from jax.experimental.pallas import tpu_sc as plsc

_LEVELS = (8, 8, 8, 6, 5)
_CD = 5
_LANES = 128
_BM = 2048
_SUB = 256

_vals = [np.linspace(-0.5, 0.5, lv).astype(np.float32) if lv % 2 else
         (np.arange(lv) / lv - 0.5).astype(np.float32) for lv in _LEVELS]
_LO = tuple(float(v[0]) for v in _vals)
_STEP = tuple(float(v[1] - v[0]) for v in _vals)
_INV = tuple(float(1.0 / (v[1] - v[0])) for v in _vals)
_MAXI = tuple(float(lv - 1) for lv in _LEVELS)
_BASIS = tuple(int(x) for x in
               np.concatenate([[1], np.cumprod(_LEVELS[:-1])]))
_HALF = tuple(int(lv) // 2 for lv in _LEVELS)
_CA = tuple(float(2 * h * bs) for h, bs in zip(_HALF, _BASIS))
_CB = tuple(float(h * bs) for h, bs in zip(_HALF, _BASIS))


def _tc(z_ref, win_ref, bin_ref, lo_ref, step_ref, inv_ref, maxi_ref,
        wout_ref, bout_ref, out_ref, zpt_ref):
    for s in range(_BM // _SUB):
        rows = pl.ds(s * _SUB, _SUB)
        zp = jnp.dot(z_ref[rows, :], win_ref[...],
                     preferred_element_type=jnp.float32) + bin_ref[...]
        for i in range(_CD):
            zpt_ref[i:i + 1, rows] = zp[:, i:i + 1].reshape(1, _SUB)
        k = jnp.clip(jnp.round((zp - lo_ref[...]) * inv_ref[...]),
                     0.0, maxi_ref[...])
        q = lo_ref[...] + k * step_ref[...]
        qb = [jnp.broadcast_to(q[:, i:i + 1], (_SUB, _LANES))
              for i in range(_CD)]
        for c in range(out_ref.shape[1] // _LANES):
            cols = pl.ds(c * _LANES, _LANES)
            acc = jnp.broadcast_to(bout_ref[0:1, cols], (_SUB, _LANES))
            for i in range(_CD):
                acc = acc + qb[i] * wout_ref[i:i + 1, cols]
            out_ref[rows, cols] = acc


def _sc_make(m):
    info = plsc.get_sparse_core_info()
    nc, ns = info.num_cores, info.num_subcores
    nw = nc * ns
    cpw = m // nw
    mesh = plsc.VectorSubcoreMesh(core_axis_name="c", subcore_axis_name="s")

    @functools.partial(
        pl.kernel, mesh=mesh,
        out_type=[
            jax.ShapeDtypeStruct((m,), jnp.float32),
            jax.ShapeDtypeStruct((nw, 16), jnp.float32),
        ],
        scratch_types=[
            pltpu.VMEM((8, cpw), jnp.float32),
            pltpu.VMEM((cpw,), jnp.float32),
            pltpu.VMEM((16,), jnp.float32),
        ],
    )
    def enc(zpt_hbm, idx_hbm, lp_hbm, zp_v, idx_v, ls_v):
        wid = lax.axis_index("s") * nc + lax.axis_index("c")
        base = wid * cpw
        pltpu.sync_copy(zpt_hbm.at[:, pl.ds(base, cpw)], zp_v)
        ls = jnp.zeros((16,), jnp.float32)
        for j in range(cpw // 16):
            sl = pl.ds(j * 16, 16)
            ia = jnp.zeros((16,), jnp.float32)
            for i in range(_CD):
                v = zp_v.at[i][sl]
                t = (v - _LO[i]) * _INV[i]
                t = jnp.clip(t, 0.0, _MAXI[i])
                k = (t + 0.5).astype(jnp.int32).astype(jnp.float32)
                q = _LO[i] + k * _STEP[i]
                ia = ia + (q * _CA[i] + _CB[i])
                e = v - q
                ls = ls + e * e
            idx_v[sl] = ia
        ls_v[...] = ls
        pltpu.sync_copy(idx_v, idx_hbm.at[pl.ds(base, cpw)])
        pltpu.sync_copy(ls_v, lp_hbm.at[wid])

    return enc


def kernel(z, W_in, b_in, W_out, b_out, v0, v1, v2, v3, v4):
    b, n, dim = z.shape
    m = b * n
    cd = _CD
    nblk = m // _BM

    win_p = jnp.zeros((dim, _LANES), jnp.float32).at[:, :cd].set(W_in.T)
    wout_p = jnp.zeros((8, dim), jnp.float32).at[:cd, :].set(W_out.T)
    bin_p = jnp.zeros((1, _LANES), jnp.float32).at[0, :cd].set(b_in)
    bout_p = b_out.reshape(1, dim)

    lo_np = np.zeros((1, _LANES), np.float32)
    st_np = np.zeros((1, _LANES), np.float32)
    iv_np = np.zeros((1, _LANES), np.float32)
    mx_np = np.zeros((1, _LANES), np.float32)
    for i in range(cd):
        lo_np[0, i] = _LO[i]
        st_np[0, i] = _STEP[i]
        iv_np[0, i] = _INV[i]
        mx_np[0, i] = _MAXI[i]

    zf = z.reshape(m, dim)
    full = lambda i: (0, 0)
    out, zpt = pl.pallas_call(
        _tc,
        grid=(nblk,),
        in_specs=[
            pl.BlockSpec((_BM, dim), lambda i: (i, 0)),
            pl.BlockSpec((dim, _LANES), full),
            pl.BlockSpec((1, _LANES), full),
            pl.BlockSpec((1, _LANES), full),
            pl.BlockSpec((1, _LANES), full),
            pl.BlockSpec((1, _LANES), full),
            pl.BlockSpec((1, _LANES), full),
            pl.BlockSpec((8, dim), full),
            pl.BlockSpec((1, dim), full),
        ],
        out_specs=[
            pl.BlockSpec((_BM, dim), lambda i: (i, 0)),
            pl.BlockSpec((8, _BM), lambda i: (0, i)),
        ],
        out_shape=[
            jax.ShapeDtypeStruct((m, dim), jnp.float32),
            jax.ShapeDtypeStruct((8, m), jnp.float32),
        ],
        compiler_params=pltpu.CompilerParams(
            dimension_semantics=("parallel",)),
    )(zf, win_p, bin_p, jnp.asarray(lo_np), jnp.asarray(st_np),
      jnp.asarray(iv_np), jnp.asarray(mx_np), wout_p, bout_p)

    idx, lpart = _sc_make(m)(zpt)

    out = out.reshape(b, n, dim)
    indices = idx.reshape(b, n)
    loss_val = jnp.sum(lpart) * (0.2 / (m * cd))
    return out, indices, loss_val
